# baseline (speedup 1.0000x reference)
import functools

import jax
import jax.numpy as jnp
from jax import lax
from jax.experimental import pallas as pl
from jax.experimental.pallas import tpu as pltpu
from jax.experimental.pallas import tpu_sc as plsc

_BATCH = 4096
_STATE = 32
_DICT = 8192
_EMB = 64

_NC = 2
_NS = 16
_NW = _NC * _NS
_L = 16

_EBLK = 4
_NBLK = _EMB // _EBLK
_VPB = _BATCH // _L
_UNROLL = 16


def _gather_body(emb_hbm, idx_hbm, out_hbm, idx_v, rows, outs,
                 sr0, sr1, so0, so1):
    w = lax.axis_index("s") * _NC + lax.axis_index("c")
    pltpu.sync_copy(idx_hbm.at[pl.ds(w, 1)], idx_v)

    sems_r = (sr0, sr1)
    sems_o = (so0, so1)

    def stage(blk):
        d = pltpu.make_async_copy(
            emb_hbm.at[pl.ds(w, 1), pl.ds(blk * _EBLK, _EBLK)],
            rows[blk % 2], sems_r[blk % 2])
        d.start()
        return d

    def flush(blk):
        d = pltpu.make_async_copy(
            outs[blk % 2],
            out_hbm.at[pl.ds(w, 1), pl.ds(blk * _EBLK, _EBLK)],
            sems_o[blk % 2])
        d.start()
        return d

    zero16 = jnp.zeros((_L,), jnp.int32)
    e_vecs = [jnp.full((_L,), e, jnp.int32) for e in range(_EBLK)]

    stage_d = {0: stage(0)}
    flush_d = {}
    for blk in range(_NBLK):
        p = blk % 2
        if blk + 1 < _NBLK:
            stage_d[blk + 1] = stage(blk + 1)
        stage_d.pop(blk).wait()
        if blk - 2 >= 0:
            flush_d.pop(blk - 2).wait()

        row_blk = rows[p]
        out_blk = outs[p]

        @plsc.parallel_loop(0, _VPB, step=1, unroll=_UNROLL)
        def _loop(v):
            off = v * _L
            idx16 = idx_v[0, pl.ds(off, _L)]
            for e in range(_EBLK):
                out_blk[0, e, pl.ds(off, _L)] = plsc.load_gather(
                    row_blk, [zero16, e_vecs[e], idx16])
        flush_d[blk] = flush(blk)
    for blk in sorted(flush_d):
        flush_d[blk].wait()


@functools.partial(
    pl.kernel,
    mesh=plsc.VectorSubcoreMesh(core_axis_name="c", subcore_axis_name="s"),
    out_type=jax.ShapeDtypeStruct((_STATE, _EMB, _BATCH), jnp.float32),
    compiler_params=pltpu.CompilerParams(needs_layout_passes=False,
                                         disable_bounds_checks=True,
                                         disable_semaphore_checks=True),
    scratch_types=[
        pltpu.VMEM((1, _BATCH), jnp.int32),
        pltpu.VMEM((1, _EBLK, _DICT), jnp.float32),
        pltpu.VMEM((1, _EBLK, _DICT), jnp.float32),
        pltpu.VMEM((1, _EBLK, _BATCH), jnp.float32),
        pltpu.VMEM((1, _EBLK, _BATCH), jnp.float32),
        pltpu.SemaphoreType.DMA,
        pltpu.SemaphoreType.DMA,
        pltpu.SemaphoreType.DMA,
        pltpu.SemaphoreType.DMA,
    ],
)
def _gather(emb_hbm, idx_hbm, out_hbm, idx_v, rows0, rows1, outs0, outs1,
            sr0, sr1, so0, so1):
    _gather_body(emb_hbm, idx_hbm, out_hbm, idx_v, (rows0, rows1),
                 (outs0, outs1), sr0, sr1, so0, so1)


def kernel(indices, embedding, ema_cluster_size, ema_w):
    del ema_cluster_size, ema_w
    emb_t = embedding.transpose(0, 2, 1)
    idx_t = indices.transpose(1, 0)
    out_t = _gather(emb_t, idx_t)
    return out_t.transpose(2, 0, 1)


# speedup vs baseline: 1.0844x; 1.0844x over previous
import functools

import jax
import jax.numpy as jnp
from jax import lax
from jax.experimental import pallas as pl
from jax.experimental.pallas import tpu as pltpu
from jax.experimental.pallas import tpu_sc as plsc

_BATCH = 4096
_STATE = 32
_DICT = 8192
_EMB = 64

_NC = 2
_NS = 16
_NW = _NC * _NS
_L = 16

_EBLK = 4
_NBLK = _EMB // _EBLK
_VPB = _BATCH // _L
_UNROLL = 8


def _gather_body(emb_hbm, idx_hbm, out_hbm, idx_v, rows, outs,
                 sr0, sr1, so0, so1):
    w = lax.axis_index("s") * _NC + lax.axis_index("c")
    pltpu.sync_copy(idx_hbm.at[pl.ds(w, 1)], idx_v)

    sems_r = (sr0, sr1)
    sems_o = (so0, so1)

    def stage(blk):
        d = pltpu.make_async_copy(
            emb_hbm.at[pl.ds(w, 1), pl.ds(blk * _EBLK, _EBLK)],
            rows[blk % 2], sems_r[blk % 2])
        d.start()
        return d

    def flush(blk):
        d = pltpu.make_async_copy(
            outs[blk % 2],
            out_hbm.at[pl.ds(w, 1), pl.ds(blk * _EBLK, _EBLK)],
            sems_o[blk % 2])
        d.start()
        return d

    zero16 = jnp.zeros((_L,), jnp.int32)
    e_vecs = [jnp.full((_L,), e, jnp.int32) for e in range(_EBLK)]

    stage_d = {0: stage(0)}
    flush_d = {}
    for blk in range(_NBLK):
        p = blk % 2
        if blk + 1 < _NBLK:
            stage_d[blk + 1] = stage(blk + 1)
        stage_d.pop(blk).wait()
        if blk - 2 >= 0:
            flush_d.pop(blk - 2).wait()

        row_blk = rows[p]
        out_blk = outs[p]

        @plsc.parallel_loop(0, _VPB, step=1, unroll=_UNROLL)
        def _loop(v):
            off = v * _L
            idx16 = idx_v[0, pl.ds(off, _L)]
            for e in range(_EBLK):
                out_blk[0, e, pl.ds(off, _L)] = plsc.load_gather(
                    row_blk, [zero16, e_vecs[e], idx16])
        flush_d[blk] = flush(blk)
    for blk in sorted(flush_d):
        flush_d[blk].wait()


@functools.partial(
    pl.kernel,
    mesh=plsc.VectorSubcoreMesh(core_axis_name="c", subcore_axis_name="s"),
    out_type=jax.ShapeDtypeStruct((_STATE, _EMB, _BATCH), jnp.float32),
    compiler_params=pltpu.CompilerParams(needs_layout_passes=False,
                                         disable_bounds_checks=True,
                                         disable_semaphore_checks=True),
    scratch_types=[
        pltpu.VMEM((1, _BATCH), jnp.int32),
        pltpu.VMEM((1, _EBLK, _DICT), jnp.float32),
        pltpu.VMEM((1, _EBLK, _DICT), jnp.float32),
        pltpu.VMEM((1, _EBLK, _BATCH), jnp.float32),
        pltpu.VMEM((1, _EBLK, _BATCH), jnp.float32),
        pltpu.SemaphoreType.DMA,
        pltpu.SemaphoreType.DMA,
        pltpu.SemaphoreType.DMA,
        pltpu.SemaphoreType.DMA,
    ],
)
def _gather(emb_hbm, idx_hbm, out_hbm, idx_v, rows0, rows1, outs0, outs1,
            sr0, sr1, so0, so1):
    _gather_body(emb_hbm, idx_hbm, out_hbm, idx_v, (rows0, rows1),
                 (outs0, outs1), sr0, sr1, so0, so1)


def kernel(indices, embedding, ema_cluster_size, ema_w):
    del ema_cluster_size, ema_w
    emb_t = embedding.transpose(0, 2, 1)
    idx_t = indices.transpose(1, 0)
    out_t = _gather(emb_t, idx_t)
    return out_t.transpose(2, 0, 1)


# speedup vs baseline: 1.1028x; 1.0170x over previous
import functools

import jax
import jax.numpy as jnp
from jax import lax
from jax.experimental import pallas as pl
from jax.experimental.pallas import tpu as pltpu
from jax.experimental.pallas import tpu_sc as plsc

_BATCH = 4096
_STATE = 32
_DICT = 8192
_EMB = 64

_NC = 2
_NS = 16
_NW = _NC * _NS
_L = 16

_EBLK = 4
_NBLK = _EMB // _EBLK
_VPB = _BATCH // _L
_UNROLL = 8


def _gather_body(emb_hbm, idx_hbm, out_hbm, idx_v, rows, outs,
                 sr0, sr1, so0, so1):
    w = lax.axis_index("s") * _NC + lax.axis_index("c")

    sems_r = (sr0, sr1)
    sems_o = (so0, so1)

    d_idx = pltpu.make_async_copy(idx_hbm.at[pl.ds(w, 1)], idx_v, so0)
    d_idx.start()

    def stage(blk):
        d = pltpu.make_async_copy(
            emb_hbm.at[pl.ds(w, 1), pl.ds(blk * _EBLK, _EBLK)],
            rows[blk % 2], sems_r[blk % 2])
        d.start()
        return d

    def flush(blk):
        d = pltpu.make_async_copy(
            outs[blk % 2],
            out_hbm.at[pl.ds(w, 1), pl.ds(blk * _EBLK, _EBLK)],
            sems_o[blk % 2])
        d.start()
        return d

    zero16 = jnp.zeros((_L,), jnp.int32)
    e_vecs = [jnp.full((_L,), e, jnp.int32) for e in range(_EBLK)]

    stage_d = {0: stage(0)}
    flush_d = {}
    for blk in range(_NBLK):
        p = blk % 2
        if blk + 1 < _NBLK:
            stage_d[blk + 1] = stage(blk + 1)
        stage_d.pop(blk).wait()
        if blk == 0:
            d_idx.wait()
        if blk - 2 >= 0:
            flush_d.pop(blk - 2).wait()

        row_blk = rows[p]
        out_blk = outs[p]

        @plsc.parallel_loop(0, _VPB, step=1, unroll=_UNROLL)
        def _loop(v):
            off = v * _L
            idx16 = idx_v[0, pl.ds(off, _L)]
            for e in range(_EBLK):
                out_blk[0, e, pl.ds(off, _L)] = plsc.load_gather(
                    row_blk, [zero16, e_vecs[e], idx16])
        flush_d[blk] = flush(blk)
    for blk in sorted(flush_d):
        flush_d[blk].wait()


@functools.partial(
    pl.kernel,
    mesh=plsc.VectorSubcoreMesh(core_axis_name="c", subcore_axis_name="s"),
    out_type=jax.ShapeDtypeStruct((_STATE, _EMB, _BATCH), jnp.float32),
    compiler_params=pltpu.CompilerParams(needs_layout_passes=False,
                                         disable_bounds_checks=True,
                                         disable_semaphore_checks=True),
    scratch_types=[
        pltpu.VMEM((1, _BATCH), jnp.int32),
        pltpu.VMEM((1, _EBLK, _DICT), jnp.float32),
        pltpu.VMEM((1, _EBLK, _DICT), jnp.float32),
        pltpu.VMEM((1, _EBLK, _BATCH), jnp.float32),
        pltpu.VMEM((1, _EBLK, _BATCH), jnp.float32),
        pltpu.SemaphoreType.DMA,
        pltpu.SemaphoreType.DMA,
        pltpu.SemaphoreType.DMA,
        pltpu.SemaphoreType.DMA,
    ],
)
def _gather(emb_hbm, idx_hbm, out_hbm, idx_v, rows0, rows1, outs0, outs1,
            sr0, sr1, so0, so1):
    _gather_body(emb_hbm, idx_hbm, out_hbm, idx_v, (rows0, rows1),
                 (outs0, outs1), sr0, sr1, so0, so1)


def kernel(indices, embedding, ema_cluster_size, ema_w):
    del ema_cluster_size, ema_w
    emb_t = embedding.transpose(0, 2, 1)
    idx_t = indices.transpose(1, 0)
    out_t = _gather(emb_t, idx_t)
    return out_t.transpose(2, 0, 1)
